# Initial kernel scaffold; baseline (speedup 1.0000x reference)
import jax
import jax.numpy as jnp
from jax.experimental import pallas as pl


def kernel(x, edge_index, edge_attr, W1, b1, W2, b2):
    raise NotImplementedError("write your pallas kernel here")


# speedup vs baseline: 4.1574x; 4.1574x over previous
import functools

import jax
import jax.numpy as jnp
from jax import lax
from jax.experimental import pallas as pl
from jax.experimental.pallas import tpu as pltpu
from jax.experimental.pallas import tpu_sc as plsc

_N = 10000
_F = 128
_NCLASS = 40
_DPAD = 64
_E = 320000
_CHUNK = 128
_NC = 2
_NS = 16
_NW = _NC * _NS
_CPT = -(-_E // (_NW * _CHUNK))
_EPAD = _NW * _CHUNK * _CPT
_NPAD = 10240
_RPT = _NPAD // _NS


def _make_sc_pass(D):
    mesh = plsc.VectorSubcoreMesh(core_axis_name="c", subcore_axis_name="s")

    @functools.partial(
        pl.kernel,
        mesh=mesh,
        compiler_params=pltpu.CompilerParams(use_tc_tiling_on_sc=False),
        out_type=jax.ShapeDtypeStruct((_NC, _NPAD, D), jnp.float32),
        scratch_types=[
            pltpu.VMEM_SHARED((_NPAD, D), jnp.float32),
            pltpu.VMEM((_CPT, _CHUNK), jnp.int32),
            pltpu.VMEM((_CPT, _CHUNK), jnp.int32),
            pltpu.VMEM((_CPT, _CHUNK), jnp.float32),
            pltpu.VMEM((_CHUNK, D), jnp.float32),
            pltpu.SemaphoreType.DMA,
        ],
    )
    def sc_pass(h_hbm, src_hbm, dst_hbm, ew_hbm, zero_hbm, out_hbm,
                acc, srcv, dstv, ewv, rows, sem):
        c = lax.axis_index("c")
        s = lax.axis_index("s")
        wid = s * _NC + c
        pltpu.sync_copy(src_hbm.at[wid], srcv)
        pltpu.sync_copy(dst_hbm.at[wid], dstv)
        pltpu.sync_copy(ew_hbm.at[wid], ewv)
        pltpu.sync_copy(zero_hbm, acc.at[pl.ds(s * _RPT, _RPT)])
        plsc.subcore_barrier()

        def chunk_body(j, carry):
            pltpu.async_copy(h_hbm.at[srcv.at[j]], rows, sem).wait()

            def grp_body(g, carry2):
                ewg = ewv[j, pl.ds(g * 16, 16)]
                for l in range(16):
                    ewb = lax.gather(
                        ewg, jnp.full((16, 1), l, jnp.int32),
                        lax.GatherDimensionNumbers(
                            offset_dims=(), collapsed_slice_dims=(0,),
                            start_index_map=(0,)),
                        slice_sizes=(1,),
                        mode=lax.GatherScatterMode.PROMISE_IN_BOUNDS)
                    r = g * 16 + l
                    for f in range(D // 16):
                        sl = pl.ds(f * 16, 16)
                        rows[r, sl] = rows[r, sl] * ewb
                return carry2

            lax.fori_loop(0, _CHUNK // 16, grp_body, 0)
            pltpu.sync_copy(rows, acc.at[dstv.at[j]], add=True)
            return carry

        lax.fori_loop(0, _CPT, chunk_body, 0)
        plsc.subcore_barrier()
        pltpu.sync_copy(acc.at[pl.ds(s * _RPT, _RPT)],
                        out_hbm.at[c, pl.ds(s * _RPT, _RPT)])

    return sc_pass


_sc_pass_128 = _make_sc_pass(_F)
_sc_pass_64 = _make_sc_pass(_DPAD)


def _tc_matmul(x, w):
    def body(x_ref, w_ref, o_ref):
        o_ref[...] = jnp.dot(x_ref[...], w_ref[...],
                             preferred_element_type=jnp.float32)

    return pl.pallas_call(
        body,
        out_shape=jax.ShapeDtypeStruct((x.shape[0], w.shape[1]), jnp.float32),
    )(x, w)


def _tc_mid(parts, b1, w2p):
    def body(p_ref, b1_ref, w_ref, o_ref):
        t = p_ref[0, :_N] + p_ref[1, :_N] + b1_ref[...]
        cm = jnp.mean(t, axis=0, keepdims=True)
        xc = t - cm
        ms = jnp.sum(xc * xc) / _N
        inv = lax.rsqrt(ms + 1e-6)
        h = jnp.maximum(xc * inv, 0.0)
        o_ref[...] = jnp.dot(h, w_ref[...], preferred_element_type=jnp.float32)

    return pl.pallas_call(
        body,
        out_shape=jax.ShapeDtypeStruct((_N, _DPAD), jnp.float32),
    )(parts, b1.reshape(1, -1), w2p)


def _tc_final(parts, b2):
    def body(q_ref, b2_ref, o_ref):
        ssum = q_ref[0, :_N] + q_ref[1, :_N]
        o_ref[...] = ssum[:, :_NCLASS] + b2_ref[...]

    return pl.pallas_call(
        body,
        out_shape=jax.ShapeDtypeStruct((_N, _NCLASS), jnp.float32),
    )(parts, b2.reshape(1, -1))


def kernel(x, edge_index, edge_attr, W1, b1, W2, b2):
    src = edge_index[0].astype(jnp.int32)
    dst = edge_index[1].astype(jnp.int32)
    ew = edge_attr.astype(jnp.float32)
    pad = _EPAD - _E
    src2 = jnp.concatenate([src, jnp.zeros((pad,), jnp.int32)]
                           ).reshape(_NW, _CPT, _CHUNK)
    dst2 = jnp.concatenate([dst, jnp.zeros((pad,), jnp.int32)]
                           ).reshape(_NW, _CPT, _CHUNK)
    ew2 = jnp.concatenate([ew, jnp.zeros((pad,), jnp.float32)]
                          ).reshape(_NW, _CPT, _CHUNK)
    zeros_f = jnp.zeros((_RPT, _F), jnp.float32)
    zeros_d = jnp.zeros((_RPT, _DPAD), jnp.float32)
    w2p = jnp.pad(W2, ((0, 0), (0, _DPAD - _NCLASS)))

    h1 = _tc_matmul(x, W1)
    p1 = _sc_pass_128(h1, src2, dst2, ew2, zeros_f)
    p = _tc_mid(p1, b1, w2p)
    p2 = _sc_pass_64(p, src2, dst2, ew2, zeros_d)
    return _tc_final(p2, b2)
